# baseline (speedup 1.0000x reference)
import functools

import jax
import jax.numpy as jnp
from jax import lax
from jax.experimental import pallas as pl
from jax.experimental.pallas import tpu as pltpu
from jax.experimental.pallas import tpu_sc as plsc

NC = 2
NS = 16
NW = NC * NS
CH = 40
R = 2
NBUF = 2


def _make_gather(n1, n2, V, D):
    rows_pw = n1 // NW
    n_groups = rows_pw // R
    n_chunks = n2 // CH
    mesh = plsc.VectorSubcoreMesh(
        core_axis_name="c", subcore_axis_name="s", num_cores=NC, num_subcores=NS
    )

    @functools.partial(
        pl.kernel,
        out_type=jax.ShapeDtypeStruct((n1, n2, D), jnp.float32),
        mesh=mesh,
        compiler_params=pltpu.CompilerParams(use_tc_tiling_on_sc=False),
        scratch_types=[
            pltpu.VMEM((rows_pw, n2), jnp.int32),
            pltpu.VMEM((NBUF, R, n2, D), jnp.float32),
            pltpu.SemaphoreType.DMA,
            pltpu.SemaphoreType.DMA,
            pltpu.SemaphoreType.DMA,
            pltpu.SemaphoreType.DMA,
        ],
    )
    def gather_kernel(idx_hbm, table_hbm, out_hbm, idx_v, rows_v, g0, g1, w0, w1):
        gsem = [g0, g1]
        wsem = [w0, w1]
        wid = lax.axis_index("s") * NC + lax.axis_index("c")
        base = wid * rows_pw
        pltpu.sync_copy(idx_hbm.at[pl.ds(base, rows_pw)], idx_v)

        def fire(g, b):
            for r in range(R):
                for j in range(n_chunks):
                    pltpu.async_copy(
                        table_hbm.at[idx_v.at[g * R + r, pl.ds(j * CH, CH)]],
                        rows_v.at[b, r, pl.ds(j * CH, CH)],
                        gsem[b],
                    )

        def drain_gathers(b):
            pltpu.make_async_copy(
                out_hbm.at[pl.ds(0, R)], rows_v.at[b], gsem[b]
            ).wait()

        def start_write(g, b):
            pltpu.async_copy(
                rows_v.at[b], out_hbm.at[pl.ds(base + g * R, R)], wsem[b]
            )

        def wait_write(b):
            pltpu.make_async_copy(
                rows_v.at[b], out_hbm.at[pl.ds(0, R)], wsem[b]
            ).wait()

        def do_group(g, b):
            drain_gathers(b)
            start_write(g, b)
            gn = g + 1
            bn = b ^ 1

            @pl.when(gn < n_groups)
            def _():
                @pl.when(gn >= NBUF)
                def _():
                    wait_write(bn)

                fire(gn, bn)

        fire(0, 0)

        def body(t, carry):
            do_group(t * NBUF, 0)
            do_group(t * NBUF + 1, 1)
            return carry

        lax.fori_loop(0, n_groups // NBUF, body, 0)
        wait_write(0)
        wait_write(1)

    return gather_kernel


def kernel(indices, emb_dim, table):
    n1, n2 = indices.shape
    V, D = table.shape
    assert n1 % (NW * R * NBUF) == 0 and n2 % CH == 0

    gate = jnp.where(
        jnp.asarray(emb_dim) == D, jnp.float32(1.0), jnp.float32(jnp.nan)
    ).astype(table.dtype)
    table_gated = (table * gate).astype(jnp.float32)

    return _make_gather(n1, n2, V, D)(indices, table_gated)
